# baseline (speedup 1.0000x reference)
import functools

import jax
import jax.numpy as jnp
from jax.experimental import pallas as pl
from jax.experimental.pallas import tpu as pltpu

H = 64
SLOTS = 8
SEQ = 24
V = 64


def _fused_body(seqs_ref, query_ref, embedT_ref, pg_w1T_ref, pg_b1_ref,
                pg_w2T_ref, rh_w1aT_ref, rh_w1bT_ref, rh_b1_ref, rh_w2_ref,
                rh_b2_ref, out_ref):
    T = seqs_ref.shape[0]
    embedT = embedT_ref[...]

    hscT = jnp.maximum(
        jnp.dot(pg_w1T_ref[...], embedT, preferred_element_type=jnp.float32)
        + pg_b1_ref[...], 0.0)
    stab_row = jnp.dot(pg_w2T_ref[...], hscT,
                       preferred_element_type=jnp.float32)

    seqsT = seqs_ref[...].T
    vid = jax.lax.broadcasted_iota(jnp.int32, (V, T), 0)

    def tok_onehot(row):
        return (vid == row).astype(jnp.float32)

    def tok_score(row):
        return jnp.dot(stab_row, tok_onehot(row),
                       preferred_element_type=jnp.float32)

    tok = seqsT[0:SLOTS, :]
    sc = jnp.concatenate([tok_score(seqsT[t:t + 1, :]) for t in range(SLOTS)],
                         axis=0)

    sidx = jax.lax.broadcasted_iota(jnp.int32, (SLOTS, T), 0)
    one = jnp.float32(1.0)

    for t in range(SLOTS, SEQ - 1):
        r0 = sc[0:1, :]
        r1 = sc[1:2, :]
        c0 = jnp.sum(jnp.where(sc[1:, :] > r0, one, 0.0), axis=0,
                     keepdims=True)
        c1 = (jnp.sum(jnp.where(sc[2:, :] > r1, one, 0.0), axis=0,
                      keepdims=True)
              + jnp.where(r0 >= r1, one, 0.0))
        evict = jnp.where(c0 >= 2.0, 0,
                          jnp.where(c1 >= 2.0, 1, 2))
        nrow = seqsT[t:t + 1, :]
        nsc = tok_score(nrow)
        keep = sidx < evict
        tok = jnp.where(keep, tok, jnp.concatenate([tok[1:, :], nrow], axis=0))
        sc = jnp.where(keep, sc, jnp.concatenate([sc[1:, :], nsc], axis=0))

    counts = tok_onehot(tok[0:1, :])
    for s in range(1, SLOTS):
        counts = counts + tok_onehot(tok[s:s + 1, :])
    summaryT = jnp.dot(embedT, counts,
                       preferred_element_type=jnp.float32) * (1.0 / SLOTS)

    qembT = jnp.dot(embedT, tok_onehot(query_ref[...]),
                    preferred_element_type=jnp.float32)

    hT = jnp.maximum(
        jnp.dot(rh_w1aT_ref[...], qembT, preferred_element_type=jnp.float32)
        + jnp.dot(rh_w1bT_ref[...], summaryT,
                  preferred_element_type=jnp.float32)
        + rh_b1_ref[...], 0.0)
    logitsT = (jnp.dot(rh_w2_ref[...], hT,
                       preferred_element_type=jnp.float32)
               + rh_b2_ref[...])
    out_ref[...] = logitsT.T


@functools.partial(jax.jit, static_argnames=("interpret",))
def _run(seqs, queryR, embedT, pg_w1T, pg_b1c, pg_w2T, rh_w1aT, rh_w1bT,
         rh_b1c, rh_w2, rh_b2r, interpret=False):
    B = seqs.shape[0]
    TILE = 4096 if B % 4096 == 0 else B
    grid = (B // TILE,)
    full = lambda shape: pl.BlockSpec(shape, lambda i: (0, 0))
    return pl.pallas_call(
        _fused_body,
        grid=grid,
        in_specs=[
            pl.BlockSpec((TILE, SEQ), lambda i: (i, 0)),
            pl.BlockSpec((1, TILE), lambda i: (0, i)),
            full((H, V)),
            full(pg_w1T.shape),
            full(pg_b1c.shape),
            full(pg_w2T.shape),
            full(rh_w1aT.shape),
            full(rh_w1bT.shape),
            full(rh_b1c.shape),
            full(rh_w2.shape),
            full(rh_b2r.shape),
        ],
        out_specs=pl.BlockSpec((TILE, V), lambda i: (i, 0)),
        out_shape=jax.ShapeDtypeStruct((B, V), jnp.float32),
        compiler_params=pltpu.CompilerParams(
            dimension_semantics=("parallel",)),
        interpret=interpret,
    )(seqs, queryR, embedT, pg_w1T, pg_b1c, pg_w2T, rh_w1aT, rh_w1bT,
      rh_b1c, rh_w2, rh_b2r)


def kernel(seqs, query_tok, embed, pg_w1, pg_b1, pg_w2, pg_b2,
           rh_w1, rh_b1, rh_w2, rh_b2, *, interpret=False):
    del pg_b2
    queryR = query_tok.astype(jnp.int32)[None, :]
    return _run(seqs.astype(jnp.int32), queryR, embed[:V].T, pg_w1.T,
                pg_b1[:, None], pg_w2.T, rh_w1[:H].T, rh_w1[H:].T,
                rh_b1[:, None], rh_w2.T, rh_b2[:, None], interpret=interpret)


# speedup vs baseline: 1.8448x; 1.8448x over previous
import functools

import jax
import jax.numpy as jnp
from jax.experimental import pallas as pl
from jax.experimental.pallas import tpu as pltpu

H = 64
SLOTS = 8
SEQ = 24
V = 64


def _fused_body(seqsT_ref, query_ref, embed_ref, pg_w1_ref, pg_b1_ref,
                pg_w2_ref, rh_w1_ref, rh_b1_ref, rh_w2_ref, rh_b2_ref,
                out_ref):
    T = seqsT_ref.shape[1]
    embed = embed_ref[...]
    embedT = embed.T

    hsc = jnp.maximum(
        jnp.dot(embed, pg_w1_ref[...], preferred_element_type=jnp.float32)
        + pg_b1_ref[...], 0.0)
    stab_row = jnp.dot(hsc, pg_w2_ref[...],
                       preferred_element_type=jnp.float32).T

    seqsT = seqsT_ref[...]
    vid = jax.lax.broadcasted_iota(jnp.int32, (V, T), 0)

    def tok_onehot(row):
        return (vid == row).astype(jnp.float32)

    def tok_score(row):
        return jnp.dot(stab_row, tok_onehot(row),
                       preferred_element_type=jnp.float32)

    tok = seqsT[0:SLOTS, :]
    sc = jnp.concatenate([tok_score(seqsT[t:t + 1, :]) for t in range(SLOTS)],
                         axis=0)

    sidx = jax.lax.broadcasted_iota(jnp.int32, (SLOTS, T), 0)
    one = jnp.float32(1.0)

    for t in range(SLOTS, SEQ - 1):
        r0 = sc[0:1, :]
        r1 = sc[1:2, :]
        c0 = jnp.sum(jnp.where(sc[1:, :] > r0, one, 0.0), axis=0,
                     keepdims=True)
        c1 = (jnp.sum(jnp.where(sc[2:, :] > r1, one, 0.0), axis=0,
                      keepdims=True)
              + jnp.where(r0 >= r1, one, 0.0))
        evict = jnp.where(c0 >= 2.0, 0,
                          jnp.where(c1 >= 2.0, 1, 2))
        nrow = seqsT[t:t + 1, :]
        nsc = tok_score(nrow)
        keep = sidx < evict
        tok = jnp.where(keep, tok, jnp.concatenate([tok[1:, :], nrow], axis=0))
        sc = jnp.where(keep, sc, jnp.concatenate([sc[1:, :], nsc], axis=0))

    counts = tok_onehot(tok[0:1, :])
    for s in range(1, SLOTS):
        counts = counts + tok_onehot(tok[s:s + 1, :])
    summaryT = jnp.dot(embedT, counts,
                       preferred_element_type=jnp.float32) * (1.0 / SLOTS)

    qembT = jnp.dot(embedT, tok_onehot(query_ref[...]),
                    preferred_element_type=jnp.float32)

    w1 = rh_w1_ref[...]
    hT = jnp.maximum(
        jnp.dot(w1[:H].T, qembT, preferred_element_type=jnp.float32)
        + jnp.dot(w1[H:].T, summaryT, preferred_element_type=jnp.float32)
        + rh_b1_ref[...].T, 0.0)
    out_ref[...] = (jnp.dot(rh_w2_ref[...].T, hT,
                            preferred_element_type=jnp.float32)
                    + rh_b2_ref[...].T)


@functools.partial(jax.jit, static_argnames=("interpret",))
def _run(seqsT, queryR, embed64, pg_w1, pg_b1r, pg_w2, rh_w1, rh_b1r,
         rh_w2, rh_b2r, interpret=False):
    B = seqsT.shape[1]
    TILE = 4096 if B % 4096 == 0 else B
    grid = (B // TILE,)
    full = lambda shape: pl.BlockSpec(shape, lambda i: (0, 0))
    return pl.pallas_call(
        _fused_body,
        grid=grid,
        in_specs=[
            pl.BlockSpec((SEQ, TILE), lambda i: (0, i)),
            pl.BlockSpec((1, TILE), lambda i: (0, i)),
            full((V, H)),
            full(pg_w1.shape),
            full(pg_b1r.shape),
            full(pg_w2.shape),
            full(rh_w1.shape),
            full(rh_b1r.shape),
            full(rh_w2.shape),
            full(rh_b2r.shape),
        ],
        out_specs=pl.BlockSpec((V, TILE), lambda i: (0, i)),
        out_shape=jax.ShapeDtypeStruct((V, B), jnp.float32),
        compiler_params=pltpu.CompilerParams(
            dimension_semantics=("parallel",)),
        interpret=interpret,
    )(seqsT, queryR, embed64, pg_w1, pg_b1r, pg_w2, rh_w1, rh_b1r,
      rh_w2, rh_b2r)


def kernel(seqs, query_tok, embed, pg_w1, pg_b1, pg_w2, pg_b2,
           rh_w1, rh_b1, rh_w2, rh_b2, *, interpret=False):
    del pg_b2
    seqsT = seqs.astype(jnp.int32).T
    queryR = query_tok.astype(jnp.int32)[None, :]
    outT = _run(seqsT, queryR, embed[:V], pg_w1, pg_b1[None, :], pg_w2,
                rh_w1, rh_b1[None, :], rh_w2, rh_b2[None, :],
                interpret=interpret)
    return outT.T
